# baseline (speedup 1.0000x reference)
import functools

import jax
import jax.numpy as jnp
from jax import lax
from jax.experimental import pallas as pl
from jax.experimental.pallas import tpu as pltpu
from jax.experimental.pallas import tpu_sc as plsc


@functools.lru_cache(maxsize=None)
def _make_select(N: int, C: int, H: int, W: int, K: int):
    info = plsc.get_sparse_core_info()
    NC, NS = info.num_cores, info.num_subcores
    NW = NC * NS
    PLANES = N * H
    PPW = PLANES // NW
    NJ = K // 16
    PC = 3

    mesh = plsc.VectorSubcoreMesh(core_axis_name="c", subcore_axis_name="s")

    @functools.partial(
        pl.kernel,
        mesh=mesh,
        compiler_params=pltpu.CompilerParams(needs_layout_passes=False),
        out_type=jax.ShapeDtypeStruct((N, H, W, K), jnp.float32),
        scratch_types=[
            pltpu.VMEM((K,), jnp.int32),
            pltpu.VMEM((2, PC, W, C), jnp.float32),
            pltpu.VMEM((2, PC, W, K), jnp.float32),
            pltpu.SemaphoreType.DMA,
            pltpu.SemaphoreType.DMA,
            pltpu.SemaphoreType.DMA,
            pltpu.SemaphoreType.DMA,
        ],
    )
    def select_kernel(x_hbm, idx_hbm, out_hbm, idxv, xbuf, obuf,
                      sg0, sg1, sw0, sw1):
        sg = (sg0, sg1)
        sw = (sw0, sw1)
        wid = lax.axis_index("s") * NC + lax.axis_index("c")
        pltpu.sync_copy(idx_hbm, idxv)
        p0 = wid * PPW

        def chunk_compute(b):
            cvecs = [idxv[pl.ds(jv * 16, 16)] for jv in range(NJ)]
            for q in range(PC):
                for w in range(W):
                    wvec = jnp.full((16,), w, jnp.int32)
                    vals = [plsc.load_gather(xbuf.at[b, q], [wvec, cvecs[jv]])
                            for jv in range(NJ)]
                    for jv in range(NJ):
                        obuf[b, q, w, pl.ds(jv * 16, 16)] = vals[jv]

        def gather_start(cp, b):
            n = cp // H
            h = cp % H
            return pltpu.async_copy(x_hbm.at[n, pl.ds(h, PC)], xbuf.at[b], sg[b])

        def write_start(cp, b):
            n = cp // H
            h = cp % H
            return pltpu.async_copy(obuf.at[b], out_hbm.at[n, pl.ds(h, PC)], sw[b])

        def drain_write(b):
            pltpu.make_async_copy(out_hbm.at[0, pl.ds(0, PC)], obuf.at[b], sw[b]).wait()

        NB = PPW // (2 * PC)
        gather_start(p0, 0)
        gather_start(p0 + PC, 1)

        def body(i, carry):
            cp = p0 + 2 * PC * i
            for b in range(2):
                pltpu.make_async_copy(
                    x_hbm.at[0, pl.ds(0, PC)], xbuf.at[b], sg[b]).wait()

                @pl.when(i > 0)
                def _():
                    drain_write(b)

                chunk_compute(b)
                write_start(cp + b * PC, b)

                @pl.when(i + 1 < NB)
                def _():
                    gather_start(cp + (b + 2) * PC, b)

            return carry

        lax.fori_loop(0, NB, body, 0)
        drain_write(0)
        drain_write(1)

    return select_kernel


def kernel(input_tensor, indexes):
    N, C, H, W = input_tensor.shape
    K = indexes.shape[0]
    if K == C:
        return input_tensor
    x_nhwc = jnp.transpose(input_tensor, (0, 2, 3, 1))
    out_nhwc = _make_select(N, C, H, W, K)(x_nhwc, indexes)
    return jnp.transpose(out_nhwc, (0, 3, 1, 2))


# speedup vs baseline: 1.1416x; 1.1416x over previous
import functools

import jax
import jax.numpy as jnp
from jax import lax
from jax.experimental import pallas as pl
from jax.experimental.pallas import tpu as pltpu
from jax.experimental.pallas import tpu_sc as plsc


@functools.lru_cache(maxsize=None)
def _make_select(N: int, C: int, H: int, W: int, K: int):
    info = plsc.get_sparse_core_info()
    NC, NS = info.num_cores, info.num_subcores
    NW = NC * NS
    PLANES = N * H
    PPW = PLANES // NW
    NJ = K // 16
    PC = 1
    NS_BUF = 4

    mesh = plsc.VectorSubcoreMesh(core_axis_name="c", subcore_axis_name="s")

    @functools.partial(
        pl.kernel,
        mesh=mesh,
        compiler_params=pltpu.CompilerParams(needs_layout_passes=False),
        out_type=jax.ShapeDtypeStruct((N, H, W, K), jnp.float32),
        scratch_types=[
            pltpu.VMEM((K,), jnp.int32),
            pltpu.VMEM((NS_BUF, PC, W, C), jnp.float32),
            pltpu.VMEM((NS_BUF, PC, W, K), jnp.float32),
            pltpu.SemaphoreType.DMA,
            pltpu.SemaphoreType.DMA,
            pltpu.SemaphoreType.DMA,
            pltpu.SemaphoreType.DMA,
            pltpu.SemaphoreType.DMA,
            pltpu.SemaphoreType.DMA,
            pltpu.SemaphoreType.DMA,
            pltpu.SemaphoreType.DMA,
        ],
    )
    def select_kernel(x_hbm, idx_hbm, out_hbm, idxv, xbuf, obuf,
                      sg0, sg1, sg2, sg3, sw0, sw1, sw2, sw3):
        sg = (sg0, sg1, sg2, sg3)
        sw = (sw0, sw1, sw2, sw3)
        wid = lax.axis_index("s") * NC + lax.axis_index("c")
        pltpu.sync_copy(idx_hbm, idxv)
        p0 = wid * PPW

        def chunk_compute(b):
            cvecs = [idxv[pl.ds(jv * 16, 16)] for jv in range(NJ)]
            for q in range(PC):
                for w in range(W):
                    wvec = jnp.full((16,), w, jnp.int32)
                    vals = [plsc.load_gather(xbuf.at[b, q], [wvec, cvecs[jv]])
                            for jv in range(NJ)]
                    for jv in range(NJ):
                        obuf[b, q, w, pl.ds(jv * 16, 16)] = vals[jv]

        def gather_start(cp, b):
            n = cp // H
            h = cp % H
            return pltpu.async_copy(x_hbm.at[n, pl.ds(h, PC)], xbuf.at[b], sg[b])

        def write_start(cp, b):
            n = cp // H
            h = cp % H
            return pltpu.async_copy(obuf.at[b], out_hbm.at[n, pl.ds(h, PC)], sw[b])

        def drain_write(b):
            pltpu.make_async_copy(out_hbm.at[0, pl.ds(0, PC)], obuf.at[b], sw[b]).wait()

        NB = PPW // (NS_BUF * PC)
        for b in range(NS_BUF):
            gather_start(p0 + b * PC, b)

        def body(i, carry):
            cp = p0 + NS_BUF * PC * i
            for b in range(NS_BUF):
                pltpu.make_async_copy(
                    x_hbm.at[0, pl.ds(0, PC)], xbuf.at[b], sg[b]).wait()

                @pl.when(i > 0)
                def _():
                    drain_write(b)

                chunk_compute(b)
                write_start(cp + b * PC, b)

                @pl.when(i + 1 < NB)
                def _():
                    gather_start(cp + (b + NS_BUF) * PC, b)

            return carry

        lax.fori_loop(0, NB, body, 0)
        for b in range(NS_BUF):
            drain_write(b)

    return select_kernel


def kernel(input_tensor, indexes):
    N, C, H, W = input_tensor.shape
    K = indexes.shape[0]
    if K == C:
        return input_tensor
    x_nhwc = jnp.transpose(input_tensor, (0, 2, 3, 1))
    out_nhwc = _make_select(N, C, H, W, K)(x_nhwc, indexes)
    return jnp.transpose(out_nhwc, (0, 3, 1, 2))


# speedup vs baseline: 1.2701x; 1.1126x over previous
import functools

import jax
import jax.numpy as jnp
from jax import lax
from jax.experimental import pallas as pl
from jax.experimental.pallas import tpu as pltpu
from jax.experimental.pallas import tpu_sc as plsc


@functools.lru_cache(maxsize=None)
def _make_select(N: int, C: int, H: int, W: int, K: int):
    info = plsc.get_sparse_core_info()
    NC, NS = info.num_cores, info.num_subcores
    NW = NC * NS
    PLANES = N * H
    PPW = PLANES // NW
    NJ = K // 16
    PW = W * C
    CB = C // 128

    mesh = plsc.VectorSubcoreMesh(core_axis_name="c", subcore_axis_name="s")

    @functools.partial(
        pl.kernel,
        mesh=mesh,
        compiler_params=pltpu.CompilerParams(needs_layout_passes=False),
        out_type=jax.ShapeDtypeStruct((N, H, W, K), jnp.float32),
        scratch_types=[
            pltpu.VMEM((K,), jnp.int32),
            pltpu.VMEM((PW,), jnp.float32),
            pltpu.VMEM((PW,), jnp.float32),
            pltpu.VMEM((2, W, K), jnp.float32),
            pltpu.SemaphoreType.DMA,
            pltpu.SemaphoreType.DMA,
            pltpu.SemaphoreType.DMA,
            pltpu.SemaphoreType.DMA,
        ],
    )
    def select_kernel(x_hbm, idx_hbm, out_hbm, idxv, xbuf0, xbuf1, obuf,
                      sg0, sg1, sw0, sw1):
        xbuf = (xbuf0, xbuf1)
        sg = (sg0, sg1)
        sw = (sw0, sw1)
        wid = lax.axis_index("s") * NC + lax.axis_index("c")
        pltpu.sync_copy(idx_hbm, idxv)
        p0 = wid * PPW

        def plane_compute(b):
            cvecs = []
            for jv in range(NJ):
                c = idxv[pl.ds(jv * 16, 16)]
                cvecs.append(((c >> 7) << 10) + (c & 127))
            for w in range(W):
                s_w = (w // 8) * (CB * 1024) + (w % 8) * 128
                vals = [plsc.load_gather(xbuf[b], [cvecs[jv] + s_w])
                        for jv in range(NJ)]
                for jv in range(NJ):
                    obuf[b, w, pl.ds(jv * 16, 16)] = vals[jv]

        def gather_start(p, b):
            return pltpu.async_copy(x_hbm.at[pl.ds(p * PW, PW)], xbuf[b], sg[b])

        def write_start(p, b):
            n = p // H
            h = p % H
            return pltpu.async_copy(obuf.at[b], out_hbm.at[n, h], sw[b])

        def drain_write(b):
            pltpu.make_async_copy(out_hbm.at[0, 0], obuf.at[b], sw[b]).wait()

        NB = PPW // 2
        gather_start(p0, 0)
        gather_start(p0 + 1, 1)

        def body(i, carry):
            p = p0 + 2 * i
            for b in range(2):
                pltpu.make_async_copy(
                    x_hbm.at[pl.ds(0, PW)], xbuf[b], sg[b]).wait()

                @pl.when(i > 0)
                def _():
                    drain_write(b)

                plane_compute(b)
                write_start(p + b, b)

                @pl.when(i + 1 < NB)
                def _():
                    gather_start(p + b + 2, b)

            return carry

        lax.fori_loop(0, NB, body, 0)
        drain_write(0)
        drain_write(1)

    return select_kernel


def kernel(input_tensor, indexes):
    N, C, H, W = input_tensor.shape
    K = indexes.shape[0]
    if K == C:
        return input_tensor
    x_nhwc = jnp.transpose(input_tensor, (0, 2, 3, 1))
    x6 = x_nhwc.reshape(N, H, W // 8, 8, C // 128, 128)
    x_flat = jnp.transpose(x6, (0, 1, 2, 4, 3, 5)).reshape(-1)
    out_nhwc = _make_select(N, C, H, W, K)(x_flat, indexes)
    return jnp.transpose(out_nhwc, (0, 3, 1, 2))
